# baseline (speedup 1.0000x reference)
import jax
import jax.numpy as jnp
from jax.experimental import pallas as pl

_K = 16


def _locse_kernel(q_ref, ct_ref, f_ref, w_ref, b_ref, o_ref):
    q = q_ref[0]
    ct = ct_ref[0]
    Q = q.shape[0]
    N = ct.shape[1]

    qsq = jnp.sum(q * q, axis=1, keepdims=True)
    csq = jnp.sum(ct * ct, axis=0, keepdims=True)
    cx = ct[0:1, :]
    cy = ct[1:2, :]
    cz = ct[2:3, :]
    qc = jax.lax.dot_general(
        q, ct, (((1,), (0,)), ((), ())),
        preferred_element_type=jnp.float32)
    d2 = qsq - 2.0 * qc + csq

    inf = jnp.float32(jnp.inf)
    nxs, nys, nzs = [], [], []
    acc = jnp.zeros(d2.shape, dtype=jnp.bool_)
    for _ in range(_K):
        m = jnp.min(jnp.where(acc, inf, d2), axis=1, keepdims=True)
        oh = (d2 <= m) & ~acc
        nxs.append(jnp.min(jnp.where(oh, cx, inf), axis=1, keepdims=True))
        nys.append(jnp.min(jnp.where(oh, cy, inf), axis=1, keepdims=True))
        nzs.append(jnp.min(jnp.where(oh, cz, inf), axis=1, keepdims=True))
        acc = acc | oh

    nbx = jnp.concatenate(nxs, axis=1)
    nby = jnp.concatenate(nys, axis=1)
    nbz = jnp.concatenate(nzs, axis=1)

    ex = jnp.broadcast_to(q[:, 0:1], (Q, _K))
    ey = jnp.broadcast_to(q[:, 1:2], (Q, _K))
    ez = jnp.broadcast_to(q[:, 2:3], (Q, _K))
    dx, dy, dz = ex - nbx, ey - nby, ez - nbz
    dist = jnp.sqrt(jnp.maximum(dx * dx + dy * dy + dz * dz, 1e-12))

    spatial = jnp.stack(
        [ex, ey, ez, nbx, nby, nbz, dx, dy, dz, dist],
        axis=0)
    spatial = spatial.reshape(10, Q * _K)

    w = w_ref[...]
    mlp = jax.lax.dot_general(
        w, spatial, (((1,), (0,)), ((), ())),
        preferred_element_type=jnp.float32) + b_ref[...]

    f = f_ref[0]
    featb = jnp.broadcast_to(f[:, :, None], f.shape + (_K,)).reshape(
        f.shape[0], Q * _K)

    o_ref[0] = jnp.concatenate([mlp, featb], axis=0)


def kernel(coords, features, W, bias):
    B, N, _ = coords.shape
    d = features.shape[1]
    d_out = W.shape[0]
    Q = 256
    ct = jnp.transpose(coords, (0, 2, 1))
    f2 = features[:, :, :, 0]
    b2 = bias[:, None]
    out = pl.pallas_call(
        _locse_kernel,
        grid=(B, N // Q),
        in_specs=[
            pl.BlockSpec((1, Q, 3), lambda b, i: (b, i, 0)),
            pl.BlockSpec((1, 3, N), lambda b, i: (b, 0, 0)),
            pl.BlockSpec((1, d, Q), lambda b, i: (b, 0, i)),
            pl.BlockSpec((d_out, 10), lambda b, i: (0, 0)),
            pl.BlockSpec((d_out, 1), lambda b, i: (0, 0)),
        ],
        out_specs=pl.BlockSpec((1, d_out + d, Q * _K), lambda b, i: (b, 0, i)),
        out_shape=jax.ShapeDtypeStruct((B, d_out + d, N * _K), jnp.float32),
    )(coords, ct, f2, W, b2)
    return out.reshape(B, d_out + d, N, _K)


# speedup vs baseline: 1.3534x; 1.3534x over previous
import jax
import jax.numpy as jnp
from jax.experimental import pallas as pl

_K = 16


def _locse_kernel(q_ref, ct_ref, f_ref, w_ref, b_ref, o_ref):
    q = q_ref[0]
    ct = ct_ref[0]
    Q = q.shape[0]
    N = ct.shape[1]

    qsq = jnp.sum(q * q, axis=1, keepdims=True)
    csq = jnp.sum(ct * ct, axis=0, keepdims=True)
    cx = ct[0:1, :]
    cy = ct[1:2, :]
    cz = ct[2:3, :]
    qc = jax.lax.dot_general(
        q, ct, (((1,), (0,)), ((), ())),
        preferred_element_type=jnp.float32)
    d2 = qsq - 2.0 * qc + csq

    inf = jnp.float32(jnp.inf)
    nxs, nys, nzs = [], [], []
    for _ in range(_K):
        m = jnp.min(d2, axis=1, keepdims=True)
        nxs.append(jnp.min(jnp.where(d2 <= m, cx, inf), axis=1, keepdims=True))
        nys.append(jnp.min(jnp.where(d2 <= m, cy, inf), axis=1, keepdims=True))
        nzs.append(jnp.min(jnp.where(d2 <= m, cz, inf), axis=1, keepdims=True))
        d2 = jnp.where(d2 <= m, inf, d2)

    nbx = jnp.concatenate(nxs, axis=1)
    nby = jnp.concatenate(nys, axis=1)
    nbz = jnp.concatenate(nzs, axis=1)

    ex = jnp.broadcast_to(q[:, 0:1], (Q, _K))
    ey = jnp.broadcast_to(q[:, 1:2], (Q, _K))
    ez = jnp.broadcast_to(q[:, 2:3], (Q, _K))
    dx, dy, dz = ex - nbx, ey - nby, ez - nbz
    dist = jnp.sqrt(jnp.maximum(dx * dx + dy * dy + dz * dz, 1e-12))

    spatial = jnp.stack(
        [ex, ey, ez, nbx, nby, nbz, dx, dy, dz, dist],
        axis=0)
    spatial = spatial.reshape(10, Q * _K)

    w = w_ref[...]
    mlp = jax.lax.dot_general(
        w, spatial, (((1,), (0,)), ((), ())),
        preferred_element_type=jnp.float32) + b_ref[...]

    f = f_ref[0]
    featb = jnp.broadcast_to(f[:, :, None], f.shape + (_K,)).reshape(
        f.shape[0], Q * _K)

    o_ref[0] = jnp.concatenate([mlp, featb], axis=0)


def kernel(coords, features, W, bias):
    B, N, _ = coords.shape
    d = features.shape[1]
    d_out = W.shape[0]
    Q = 256
    ct = jnp.transpose(coords, (0, 2, 1))
    f2 = features[:, :, :, 0]
    b2 = bias[:, None]
    out = pl.pallas_call(
        _locse_kernel,
        grid=(B, N // Q),
        in_specs=[
            pl.BlockSpec((1, Q, 3), lambda b, i: (b, i, 0)),
            pl.BlockSpec((1, 3, N), lambda b, i: (b, 0, 0)),
            pl.BlockSpec((1, d, Q), lambda b, i: (b, 0, i)),
            pl.BlockSpec((d_out, 10), lambda b, i: (0, 0)),
            pl.BlockSpec((d_out, 1), lambda b, i: (0, 0)),
        ],
        out_specs=pl.BlockSpec((1, d_out + d, Q * _K), lambda b, i: (b, 0, i)),
        out_shape=jax.ShapeDtypeStruct((B, d_out + d, N * _K), jnp.float32),
    )(coords, ct, f2, W, b2)
    return out.reshape(B, d_out + d, N, _K)


# speedup vs baseline: 1.3896x; 1.0267x over previous
import jax
import jax.numpy as jnp
from jax.experimental import pallas as pl
from jax.experimental.pallas import tpu as pltpu

_K = 16


def _locse_kernel(q_ref, ct_ref, s_ref, f_ref, w_ref, b_ref, o_ref, ext_ref):
    q = q_ref[0]
    ct = ct_ref[0]
    Q = q.shape[0]
    N = ct.shape[1]

    qsq = jnp.sum(q * q, axis=1, keepdims=True)
    csq = jnp.sum(ct * ct, axis=0, keepdims=True)
    qc = jax.lax.dot_general(
        q, ct, (((1,), (0,)), ((), ())),
        preferred_element_type=jnp.float32)
    d2 = qsq - 2.0 * qc + csq

    s = s_ref[0]
    inf = jnp.float32(jnp.inf)

    def body(i, d2):
        m = jnp.min(d2, axis=1, keepdims=True)
        sel = d2 <= m
        oh = sel.astype(jnp.bfloat16)
        e = jax.lax.dot_general(
            oh, s, (((1,), (0,)), ((), ())),
            preferred_element_type=jnp.float32)
        ext_ref[i] = e
        return jnp.where(sel, inf, d2)

    jax.lax.fori_loop(0, _K, body, d2)

    nxs, nys, nzs = [], [], []
    for i in range(_K):
        e = ext_ref[i]
        nxs.append((e[:, 0:1] + e[:, 1:2]) + e[:, 2:3])
        nys.append((e[:, 3:4] + e[:, 4:5]) + e[:, 5:6])
        nzs.append((e[:, 6:7] + e[:, 7:8]) + e[:, 8:9])
    nbx = jnp.concatenate(nxs, axis=1)
    nby = jnp.concatenate(nys, axis=1)
    nbz = jnp.concatenate(nzs, axis=1)

    ex = jnp.broadcast_to(q[:, 0:1], (Q, _K))
    ey = jnp.broadcast_to(q[:, 1:2], (Q, _K))
    ez = jnp.broadcast_to(q[:, 2:3], (Q, _K))
    dx, dy, dz = ex - nbx, ey - nby, ez - nbz
    dist = jnp.sqrt(jnp.maximum(dx * dx + dy * dy + dz * dz, 1e-12))

    spatial = jnp.stack(
        [ex, ey, ez, nbx, nby, nbz, dx, dy, dz, dist],
        axis=0)
    spatial = spatial.reshape(10, Q * _K)

    w = w_ref[...]
    mlp = jax.lax.dot_general(
        w, spatial, (((1,), (0,)), ((), ())),
        preferred_element_type=jnp.float32) + b_ref[...]

    f = f_ref[0]
    featb = jnp.broadcast_to(f[:, :, None], f.shape + (_K,)).reshape(
        f.shape[0], Q * _K)

    o_ref[0] = jnp.concatenate([mlp, featb], axis=0)


def kernel(coords, features, W, bias):
    B, N, _ = coords.shape
    d = features.shape[1]
    d_out = W.shape[0]
    Q = 128
    ct = jnp.transpose(coords, (0, 2, 1))
    f2 = features[:, :, :, 0]
    b2 = bias[:, None]
    splits = []
    for a in range(3):
        c = coords[:, :, a]
        hi = c.astype(jnp.bfloat16)
        r1 = c - hi.astype(jnp.float32)
        mid = r1.astype(jnp.bfloat16)
        lo = (r1 - mid.astype(jnp.float32)).astype(jnp.bfloat16)
        splits += [hi, mid, lo]
    zero = jnp.zeros_like(splits[0])
    s_tab = jnp.stack(splits + [zero] * 7, axis=2)
    out = pl.pallas_call(
        _locse_kernel,
        grid=(B, N // Q),
        in_specs=[
            pl.BlockSpec((1, Q, 3), lambda b, i: (b, i, 0)),
            pl.BlockSpec((1, 3, N), lambda b, i: (b, 0, 0)),
            pl.BlockSpec((1, N, 16), lambda b, i: (b, 0, 0)),
            pl.BlockSpec((1, d, Q), lambda b, i: (b, 0, i)),
            pl.BlockSpec((d_out, 10), lambda b, i: (0, 0)),
            pl.BlockSpec((d_out, 1), lambda b, i: (0, 0)),
        ],
        out_specs=pl.BlockSpec((1, d_out + d, Q * _K), lambda b, i: (b, 0, i)),
        out_shape=jax.ShapeDtypeStruct((B, d_out + d, N * _K), jnp.float32),
        scratch_shapes=[pltpu.VMEM((_K, Q, 16), jnp.float32)],
    )(coords, ct, s_tab, f2, W, b2)
    return out.reshape(B, d_out + d, N, _K)


# speedup vs baseline: 1.9871x; 1.4300x over previous
import jax
import jax.numpy as jnp
from jax.experimental import pallas as pl
from jax.experimental.pallas import tpu as pltpu

_K = 16


def _locse_kernel(q_ref, ct_ref, s_ref, f_ref, w_ref, b_ref, o_ref, ext_ref):
    q = q_ref[0]
    ct = ct_ref[0]
    Q = q.shape[0]
    N = ct.shape[1]

    qsq = jnp.sum(q * q, axis=1, keepdims=True)
    csq = jnp.sum(ct * ct, axis=0, keepdims=True)
    qc = jax.lax.dot_general(
        q, ct, (((1,), (0,)), ((), ())),
        preferred_element_type=jnp.float32)
    d2 = qsq - 2.0 * qc + csq

    s = s_ref[0]
    inf = jnp.float32(jnp.inf)

    _G = 4

    def body(g, d2):
        for j in range(_G):
            m = jnp.min(d2, axis=1, keepdims=True)
            sel = d2 <= m
            oh = sel.astype(jnp.bfloat16)
            e = jax.lax.dot_general(
                oh, s, (((1,), (0,)), ((), ())),
                preferred_element_type=jnp.float32)
            ext_ref[g * _G + j] = e
            d2 = jnp.where(sel, inf, d2)
        return d2

    jax.lax.fori_loop(0, _K // _G, body, d2)

    nxs, nys, nzs = [], [], []
    for i in range(_K):
        e = ext_ref[i]
        cnt = e[:, 9:10]
        nxs.append(((e[:, 0:1] + e[:, 1:2]) + e[:, 2:3]) / cnt)
        nys.append(((e[:, 3:4] + e[:, 4:5]) + e[:, 5:6]) / cnt)
        nzs.append(((e[:, 6:7] + e[:, 7:8]) + e[:, 8:9]) / cnt)
    nbx = jnp.concatenate(nxs, axis=1)
    nby = jnp.concatenate(nys, axis=1)
    nbz = jnp.concatenate(nzs, axis=1)

    ex = jnp.broadcast_to(q[:, 0:1], (Q, _K))
    ey = jnp.broadcast_to(q[:, 1:2], (Q, _K))
    ez = jnp.broadcast_to(q[:, 2:3], (Q, _K))
    dx, dy, dz = ex - nbx, ey - nby, ez - nbz
    dist = jnp.sqrt(jnp.maximum(dx * dx + dy * dy + dz * dz, 1e-12))

    spatial = jnp.stack(
        [ex, ey, ez, nbx, nby, nbz, dx, dy, dz, dist],
        axis=0)
    spatial = spatial.reshape(10, Q * _K)

    w = w_ref[...]
    mlp = jax.lax.dot_general(
        w, spatial, (((1,), (0,)), ((), ())),
        preferred_element_type=jnp.float32) + b_ref[...]

    f = f_ref[0]
    featb = jnp.broadcast_to(f[:, :, None], f.shape + (_K,)).reshape(
        f.shape[0], Q * _K)

    o_ref[0] = jnp.concatenate([mlp, featb], axis=0)


def kernel(coords, features, W, bias):
    B, N, _ = coords.shape
    d = features.shape[1]
    d_out = W.shape[0]
    Q = 128
    ct = jnp.transpose(coords, (0, 2, 1))
    f2 = features[:, :, :, 0]
    b2 = bias[:, None]
    splits = []
    for a in range(3):
        c = coords[:, :, a]
        hi = c.astype(jnp.bfloat16)
        r1 = c - hi.astype(jnp.float32)
        mid = r1.astype(jnp.bfloat16)
        lo = (r1 - mid.astype(jnp.float32)).astype(jnp.bfloat16)
        splits += [hi, mid, lo]
    zero = jnp.zeros_like(splits[0])
    ones = jnp.ones_like(splits[0])
    s_tab = jnp.stack(splits + [ones] + [zero] * 6, axis=2)
    out = pl.pallas_call(
        _locse_kernel,
        grid=(B, N // Q),
        in_specs=[
            pl.BlockSpec((1, Q, 3), lambda b, i: (b, i, 0)),
            pl.BlockSpec((1, 3, N), lambda b, i: (b, 0, 0)),
            pl.BlockSpec((1, N, 16), lambda b, i: (b, 0, 0)),
            pl.BlockSpec((1, d, Q), lambda b, i: (b, 0, i)),
            pl.BlockSpec((d_out, 10), lambda b, i: (0, 0)),
            pl.BlockSpec((d_out, 1), lambda b, i: (0, 0)),
        ],
        out_specs=pl.BlockSpec((1, d_out + d, Q * _K), lambda b, i: (b, 0, i)),
        out_shape=jax.ShapeDtypeStruct((B, d_out + d, N * _K), jnp.float32),
        scratch_shapes=[pltpu.VMEM((_K, Q, 16), jnp.float32)],
    )(coords, ct, s_tab, f2, W, b2)
    return out.reshape(B, d_out + d, N, _K)
